# baseline (speedup 1.0000x reference)
import functools

import jax
import jax.numpy as jnp
from jax import lax
from jax.experimental import pallas as pl
from jax.experimental.pallas import tpu as pltpu
from jax.experimental.pallas import tpu_sc as plsc

RESO = 128
NPTS = 2_000_000
BLK = 3200
NTILE = NPTS // 128
QB = BLK // 128
NBLK = NPTS // BLK
NWORKERS = 32
NT = (NBLK + NWORKERS - 1) // NWORKERS
CHUNKS = BLK // 16

_F32 = jnp.float32
_I32 = jnp.int32


def _rsqrt(s):
    i = lax.bitcast_convert_type(s, _I32)
    i = jnp.int32(0x5F3759DF) - lax.shift_right_arithmetic(i, 1)
    y = lax.bitcast_convert_type(i, _F32)
    for _ in range(3):
        y = y * (1.5 - 0.5 * s * y * y)
    return y


def _make_body(is_sdf):
    mult = 1 if is_sdf else 3

    def _body(*args):
        (pts_h, idx_h, xl_h, yl_h, zl_h, off_h), rest = args[:6], args[6:]
        out_h, rest = rest[0], rest[1:]
        (xl_v, yl_v, zl_v, off_v) = rest[:4]
        idx_b = rest[4:6]
        pts_b = rest[6:8]
        out_b = rest[8:10]
        isem0, isem1, osem0, osem1 = rest[10:]

        w = lax.axis_index("s") * 2 + lax.axis_index("c")
        pltpu.sync_copy(xl_h, xl_v)
        pltpu.sync_copy(yl_h, yl_v)
        pltpu.sync_copy(zl_h, zl_v)
        pltpu.sync_copy(off_h, off_v)

        isem = (isem0, isem1)
        osem = (osem0, osem1)
        d = off_v[0]
        e = off_v[1]
        f = off_v[2]
        g = off_v[3]

        nblk_w = (NBLK - 1 - w) // NWORKERS + 1

        def hbase(k):
            return (w + k * NWORKERS) * BLK

        def in_start(k, sl):
            base = hbase(k)
            sem = isem[sl]
            pltpu.async_copy(idx_h.at[pl.ds(base, BLK)], idx_b[sl], sem)
            pltpu.async_copy(pts_h.at[pl.ds(3 * base, 3 * BLK)], pts_b[sl], sem)

        def in_wait(sl):
            sem = isem[sl]
            pltpu.make_async_copy(idx_h.at[pl.ds(0, BLK)], idx_b[sl], sem).wait()
            pltpu.make_async_copy(pts_h.at[pl.ds(0, 3 * BLK)], pts_b[sl],
                                  sem).wait()

        def out_start(k, sl):
            base = hbase(k)
            pltpu.async_copy(out_b[sl], out_h.at[pl.ds(mult * base, mult * BLK)],
                             osem[sl])

        def out_wait(sl):
            pltpu.make_async_copy(out_b[sl], out_h.at[pl.ds(0, mult * BLK)],
                                  osem[sl]).wait()

        def decode(sl, i):
            s = i * 16
            off = (lax.shift_right_logical(i, 3) * 384
                   + lax.bitwise_and(i, 7) * 16)
            idx = idx_b[sl][pl.ds(s, 16)]
            iz = lax.bitwise_and(idx, 127)
            iy = lax.bitwise_and(lax.shift_right_logical(idx, 7), 127)
            ix = lax.shift_right_logical(idx, 14)
            a = plsc.load_gather(xl_v, [ix])
            b = plsc.load_gather(yl_v, [iy])
            c = plsc.load_gather(zl_v, [iz])
            px = pts_b[sl][pl.ds(off, 16)] + ix.astype(_F32)
            py = pts_b[sl][pl.ds(off + 128, 16)] + iy.astype(_F32)
            pz = pts_b[sl][pl.ds(off + 256, 16)] + iz.astype(_F32)
            return a, b, c, px, py, pz, off

        def compute(sl):
            if is_sdf:
                @plsc.parallel_loop(0, CHUNKS, 1, unroll=4)
                def _(i):
                    a, b, c, px, py, pz, _ = decode(sl, i)
                    val = (px * (a * px + d) + py * (b * py + e)
                           + pz * (c * pz + f) + g)
                    out_b[sl][pl.ds(i * 16, 16)] = val
            else:
                @plsc.parallel_loop(0, CHUNKS, 1, unroll=4)
                def _(i):
                    a, b, c, px, py, pz, off = decode(sl, i)
                    gx = 2.0 * a * px + d
                    gy = 2.0 * b * py + e
                    gz = 2.0 * c * pz + f
                    s2 = gx * gx + gy * gy + gz * gz
                    norm = s2 * _rsqrt(s2)
                    inv = 1.0 / (norm + 1e-8)
                    out_b[sl][pl.ds(off, 16)] = gx * inv
                    out_b[sl][pl.ds(off + 128, 16)] = gy * inv
                    out_b[sl][pl.ds(off + 256, 16)] = gz * inv

        @pl.when(nblk_w > 0)
        def _():
            in_start(0, 0)

        def pair(k2, carry):
            for b2 in (0, 1):
                k = 2 * k2 + b2

                @pl.when(k + 1 < nblk_w)
                def _():
                    in_start(k + 1, 1 - b2)

                @pl.when(k < nblk_w)
                def _():
                    in_wait(b2)

                    @pl.when(k >= 2)
                    def _():
                        out_wait(b2)

                    compute(b2)
                    out_start(k, b2)
            return carry

        lax.fori_loop(0, (NT + 1) // 2, pair, 0)
        out_wait(0)

        @pl.when(nblk_w >= 2)
        def _():
            out_wait(1)

    return _body


def _make_call(is_sdf):
    mult = 1 if is_sdf else 3
    return functools.partial(
        pl.kernel,
        out_type=[jax.ShapeDtypeStruct((mult * NPTS,), _F32)],
        mesh=plsc.VectorSubcoreMesh(core_axis_name="c", subcore_axis_name="s"),
        compiler_params=pltpu.CompilerParams(
            needs_layout_passes=False, use_tc_tiling_on_sc=False),
        scratch_types=[
            pltpu.VMEM((RESO,), _F32),
            pltpu.VMEM((RESO,), _F32),
            pltpu.VMEM((RESO,), _F32),
            pltpu.VMEM((4, 16), _F32),
        ] + [pltpu.VMEM((BLK,), _I32)] * 2
        + [pltpu.VMEM((3 * BLK,), _F32)] * 2
        + [pltpu.VMEM((mult * BLK,), _F32)] * 2
        + [
            pltpu.SemaphoreType.DMA,
            pltpu.SemaphoreType.DMA,
            pltpu.SemaphoreType.DMA,
            pltpu.SemaphoreType.DMA,
        ],
    )(_make_body(is_sdf))


_sdf_call = _make_call(True)
_nrm_call = _make_call(False)


def kernel(renderPointList, renderIndexList, sdfPointList, sdfIndexList,
           xLayer, yLayer, zLayer, offset):
    off16 = jnp.broadcast_to(offset[:, None], (4, 16))
    spts = sdfPointList.reshape(NTILE, 128, 3).transpose(0, 2, 1).reshape(-1)
    rpts = renderPointList.reshape(NTILE, 128, 3).transpose(0, 2, 1).reshape(-1)
    sdf, = _sdf_call(
        spts, sdfIndexList, xLayer, yLayer, zLayer, off16)
    nout, = _nrm_call(
        rpts, renderIndexList, xLayer, yLayer, zLayer, off16)
    nrm = nout.reshape(NTILE, 3, 128).transpose(0, 2, 1).reshape(NPTS, 3)
    return (sdf, nrm)


# speedup vs baseline: 1.0249x; 1.0249x over previous
import functools

import jax
import jax.numpy as jnp
from jax import lax
from jax.experimental import pallas as pl
from jax.experimental.pallas import tpu as pltpu
from jax.experimental.pallas import tpu_sc as plsc

RESO = 128
NPTS = 2_000_000
BLK = 3200
NTILE = NPTS // 128
QB = BLK // 128
NBLK = NPTS // BLK
NWORKERS = 32
NT = (NBLK + NWORKERS - 1) // NWORKERS
CHUNKS = BLK // 16

_F32 = jnp.float32
_I32 = jnp.int32


def _rsqrt(s):
    i = lax.bitcast_convert_type(s, _I32)
    i = jnp.int32(0x5F3759DF) - lax.shift_right_arithmetic(i, 1)
    y = lax.bitcast_convert_type(i, _F32)
    for _ in range(3):
        y = y * (1.5 - 0.5 * s * y * y)
    return y


def _make_body(is_sdf):
    mult = 1 if is_sdf else 3

    def _body(*args):
        (pts_h, idx_h, xl_h, yl_h, zl_h, off_h), rest = args[:6], args[6:]
        out_h, rest = rest[0], rest[1:]
        (xl_v, yl_v, zl_v, off_v) = rest[:4]
        idx_b = rest[4:6]
        pts_b = rest[6:8]
        out_b = rest[8:10]
        isem0, isem1, osem0, osem1 = rest[10:]

        w = lax.axis_index("s") * 2 + lax.axis_index("c")
        pltpu.sync_copy(xl_h, xl_v)
        pltpu.sync_copy(yl_h, yl_v)
        pltpu.sync_copy(zl_h, zl_v)
        pltpu.sync_copy(off_h, off_v)

        isem = (isem0, isem1)
        osem = (osem0, osem1)
        d = off_v[0]
        e = off_v[1]
        f = off_v[2]
        g = off_v[3]

        nblk_w = (NBLK - 1 - w) // NWORKERS + 1

        def hbase(k):
            return (w + k * NWORKERS) * BLK

        def in_start(k, sl):
            base = hbase(k)
            sem = isem[sl]
            pltpu.async_copy(idx_h.at[pl.ds(base, BLK)], idx_b[sl], sem)
            pltpu.async_copy(pts_h.at[pl.ds(3 * base, 3 * BLK)], pts_b[sl], sem)

        def in_wait(sl):
            sem = isem[sl]
            pltpu.make_async_copy(idx_h.at[pl.ds(0, BLK)], idx_b[sl], sem).wait()
            pltpu.make_async_copy(pts_h.at[pl.ds(0, 3 * BLK)], pts_b[sl],
                                  sem).wait()

        def out_start(k, sl):
            base = hbase(k)
            pltpu.async_copy(out_b[sl], out_h.at[pl.ds(mult * base, mult * BLK)],
                             osem[sl])

        def out_wait(sl):
            pltpu.make_async_copy(out_b[sl], out_h.at[pl.ds(0, mult * BLK)],
                                  osem[sl]).wait()

        def decode(sl, i):
            s = i * 16
            off = (lax.shift_right_logical(i, 3) * 384
                   + lax.bitwise_and(i, 7) * 16)
            idx = idx_b[sl][pl.ds(s, 16)]
            iz = lax.bitwise_and(idx, 127)
            iy = lax.bitwise_and(lax.shift_right_logical(idx, 7), 127)
            ix = lax.shift_right_logical(idx, 14)
            a = plsc.load_gather(xl_v, [ix])
            b = plsc.load_gather(yl_v, [iy])
            c = plsc.load_gather(zl_v, [iz])
            px = pts_b[sl][pl.ds(off, 16)] + ix.astype(_F32)
            py = pts_b[sl][pl.ds(off + 128, 16)] + iy.astype(_F32)
            pz = pts_b[sl][pl.ds(off + 256, 16)] + iz.astype(_F32)
            return a, b, c, px, py, pz, off

        def compute(sl):
            if is_sdf:
                @plsc.parallel_loop(0, CHUNKS, 1, unroll=8)
                def _(i):
                    a, b, c, px, py, pz, _ = decode(sl, i)
                    val = (px * (a * px + d) + py * (b * py + e)
                           + pz * (c * pz + f) + g)
                    out_b[sl][pl.ds(i * 16, 16)] = val
            else:
                @plsc.parallel_loop(0, CHUNKS, 1, unroll=8)
                def _(i):
                    a, b, c, px, py, pz, off = decode(sl, i)
                    gx = 2.0 * a * px + d
                    gy = 2.0 * b * py + e
                    gz = 2.0 * c * pz + f
                    s2 = gx * gx + gy * gy + gz * gz
                    norm = s2 * _rsqrt(s2)
                    inv = 1.0 / (norm + 1e-8)
                    out_b[sl][pl.ds(off, 16)] = gx * inv
                    out_b[sl][pl.ds(off + 128, 16)] = gy * inv
                    out_b[sl][pl.ds(off + 256, 16)] = gz * inv

        @pl.when(nblk_w > 0)
        def _():
            in_start(0, 0)

        def pair(k2, carry):
            for b2 in (0, 1):
                k = 2 * k2 + b2

                @pl.when(k + 1 < nblk_w)
                def _():
                    in_start(k + 1, 1 - b2)

                @pl.when(k < nblk_w)
                def _():
                    in_wait(b2)

                    @pl.when(k >= 2)
                    def _():
                        out_wait(b2)

                    compute(b2)
                    out_start(k, b2)
            return carry

        lax.fori_loop(0, (NT + 1) // 2, pair, 0)
        out_wait(0)

        @pl.when(nblk_w >= 2)
        def _():
            out_wait(1)

    return _body


def _make_call(is_sdf):
    mult = 1 if is_sdf else 3
    return functools.partial(
        pl.kernel,
        out_type=[jax.ShapeDtypeStruct((mult * NPTS,), _F32)],
        mesh=plsc.VectorSubcoreMesh(core_axis_name="c", subcore_axis_name="s"),
        compiler_params=pltpu.CompilerParams(
            needs_layout_passes=False, use_tc_tiling_on_sc=False),
        scratch_types=[
            pltpu.VMEM((RESO,), _F32),
            pltpu.VMEM((RESO,), _F32),
            pltpu.VMEM((RESO,), _F32),
            pltpu.VMEM((4, 16), _F32),
        ] + [pltpu.VMEM((BLK,), _I32)] * 2
        + [pltpu.VMEM((3 * BLK,), _F32)] * 2
        + [pltpu.VMEM((mult * BLK,), _F32)] * 2
        + [
            pltpu.SemaphoreType.DMA,
            pltpu.SemaphoreType.DMA,
            pltpu.SemaphoreType.DMA,
            pltpu.SemaphoreType.DMA,
        ],
    )(_make_body(is_sdf))


_sdf_call = _make_call(True)
_nrm_call = _make_call(False)


def kernel(renderPointList, renderIndexList, sdfPointList, sdfIndexList,
           xLayer, yLayer, zLayer, offset):
    off16 = jnp.broadcast_to(offset[:, None], (4, 16))
    spts = sdfPointList.reshape(NTILE, 128, 3).transpose(0, 2, 1).reshape(-1)
    rpts = renderPointList.reshape(NTILE, 128, 3).transpose(0, 2, 1).reshape(-1)
    sdf, = _sdf_call(
        spts, sdfIndexList, xLayer, yLayer, zLayer, off16)
    nout, = _nrm_call(
        rpts, renderIndexList, xLayer, yLayer, zLayer, off16)
    nrm = nout.reshape(NTILE, 3, 128).transpose(0, 2, 1).reshape(NPTS, 3)
    return (sdf, nrm)
